# baseline (speedup 1.0000x reference)
import functools

import jax
import jax.numpy as jnp
from jax import lax
from jax.experimental import pallas as pl
from jax.experimental.pallas import tpu as pltpu
from jax.experimental.pallas import tpu_sc as plsc

NC = 2
NS = 16
LANES = 16
CH = 80

_f32 = jnp.float32


def _sc_mesh():
    return plsc.VectorSubcoreMesh(core_axis_name="c", subcore_axis_name="s")


def _make_hist(n_pad, k):
    stripe = n_pad // NS
    nz = stripe // 128

    @functools.partial(
        pl.kernel,
        out_type=jax.ShapeDtypeStruct((NC * n_pad, LANES), _f32),
        mesh=_sc_mesh(),
        scratch_types=[
            pltpu.VMEM((k, CH), jnp.int32),
            pltpu.VMEM((CH, LANES), _f32),
            pltpu.VMEM((128, LANES), _f32),
            pltpu.VMEM_SHARED((n_pad, LANES), _f32),
            pltpu.SemaphoreType.DMA,
        ],
    )
    def hist(dst_hbm, out_hbm, dst_v, ones_v, z_v, acc, sem):
        cid = lax.axis_index("c")
        sid = lax.axis_index("s")
        wid = cid * NS + sid

        onehot = jnp.where(lax.iota(jnp.int32, LANES) == 0, 1.0, 0.0)
        zrow = jnp.zeros((LANES,), _f32)

        @pl.loop(0, 128)
        def _(r):
            z_v[r] = zrow

        @pl.loop(0, CH)
        def _(r):
            ones_v[r] = onehot

        for t in range(nz):
            pltpu.sync_copy(z_v, acc.at[pl.ds(sid * stripe + t * 128, 128)])
        plsc.subcore_barrier()

        pltpu.async_copy(dst_hbm.at[wid], dst_v, sem).wait()

        fire = 25
        assert k % fire == 0

        @pl.loop(0, k, step=fire)
        def _(j0):
            for t in range(fire):
                pltpu.async_copy(ones_v, acc.at[dst_v.at[j0 + t]], sem, add=True)
            for t in range(fire):
                pltpu.make_async_copy(ones_v, acc.at[dst_v.at[j0 + t]], sem).wait()

        plsc.subcore_barrier()
        pltpu.sync_copy(
            acc.at[pl.ds(sid * stripe, stripe)],
            out_hbm.at[pl.ds(cid * n_pad + sid * stripe, stripe)],
        )

    return hist


def _make_scatter(n, n_pad, d, k):
    stripe = n_pad // NS
    assert k % 2 == 1 and k >= 3

    @functools.partial(
        pl.kernel,
        out_type=jax.ShapeDtypeStruct((NC * n_pad, d), _f32),
        mesh=_sc_mesh(),
        scratch_types=[
            pltpu.VMEM((2, CH), jnp.int32),
            pltpu.VMEM((2, CH), jnp.int32),
            pltpu.VMEM((CH, d), _f32),
            pltpu.VMEM((CH, d), _f32),
            pltpu.VMEM_SHARED((n_pad, d), _f32),
            pltpu.SemaphoreType.DMA,
            pltpu.SemaphoreType.DMA,
            pltpu.SemaphoreType.DMA,
            pltpu.SemaphoreType.DMA,
        ],
    )
    def scatter(rows_hbm, idx_hbm, zeros_hbm, out_hbm,
                i0, i1, buf0, buf1, acc, isem0, isem1, sem0, sem1):
        cid = lax.axis_index("c")
        sid = lax.axis_index("s")
        wid = cid * NS + sid
        me = idx_hbm.at[wid]

        pltpu.sync_copy(zeros_hbm, acc.at[pl.ds(sid * stripe, stripe)])
        plsc.subcore_barrier()

        pltpu.sync_copy(me.at[0], i0)
        pltpu.async_copy(rows_hbm.at[i0.at[0]], buf0, sem0)
        pltpu.async_copy(me.at[1], i1, isem1)

        @pl.loop(0, k - 1, step=2)
        def _(j):
            pltpu.make_async_copy(me.at[j + 1], i1, isem1).wait()
            pltpu.async_copy(rows_hbm.at[i1.at[0]], buf1, sem1)
            pltpu.make_async_copy(rows_hbm.at[i0.at[0]], buf0, sem0).wait()
            pltpu.sync_copy(buf0, acc.at[i0.at[1]], add=True)
            pltpu.async_copy(me.at[j + 2], i0, isem0)
            pltpu.make_async_copy(me.at[j + 2], i0, isem0).wait()
            pltpu.async_copy(rows_hbm.at[i0.at[0]], buf0, sem0)
            pltpu.make_async_copy(rows_hbm.at[i1.at[0]], buf1, sem1).wait()
            pltpu.sync_copy(buf1, acc.at[i1.at[1]], add=True)

            @pl.when(j + 3 <= k - 1)
            def _():
                pltpu.async_copy(me.at[j + 3], i1, isem1)

        pltpu.make_async_copy(rows_hbm.at[i0.at[0]], buf0, sem0).wait()
        pltpu.sync_copy(buf0, acc.at[i0.at[1]], add=True)

        plsc.subcore_barrier()
        pltpu.sync_copy(
            acc.at[pl.ds(sid * stripe, stripe)],
            out_hbm.at[pl.ds(cid * n_pad + sid * stripe, stripe)],
        )

    return scatter


def _mm_body(x_ref, w_ref, o_ref):
    o_ref[...] = jnp.dot(x_ref[...], w_ref[...], preferred_element_type=_f32)


def _tc_matmul(x, w, blk):
    n, din = x.shape
    dout = w.shape[1]
    return pl.pallas_call(
        _mm_body,
        grid=(n // blk,),
        in_specs=[
            pl.BlockSpec((blk, din), lambda i: (i, 0)),
            pl.BlockSpec((din, dout), lambda i: (0, 0)),
        ],
        out_specs=pl.BlockSpec((blk, dout), lambda i: (i, 0)),
        out_shape=jax.ShapeDtypeStruct((n, dout), _f32),
    )(x, w)


def _dinv_of(ca_ref, cb_ref):
    deg = 1.0 + ca_ref[..., 0:1] + cb_ref[..., 0:1]
    return lax.rsqrt(deg)


def _pre_body(ca_ref, cb_ref, g_ref, hp_ref):
    hp_ref[...] = g_ref[...] * _dinv_of(ca_ref, cb_ref)


def _tc_pre(cnt_a, cnt_b, g, blk):
    n, d = g.shape
    return pl.pallas_call(
        _pre_body,
        grid=(n // blk,),
        in_specs=[
            pl.BlockSpec((blk, LANES), lambda i: (i, 0)),
            pl.BlockSpec((blk, LANES), lambda i: (i, 0)),
            pl.BlockSpec((blk, d), lambda i: (i, 0)),
        ],
        out_specs=pl.BlockSpec((blk, d), lambda i: (i, 0)),
        out_shape=jax.ShapeDtypeStruct((n, d), _f32),
    )(cnt_a, cnt_b, g)


def _mid_body(sa_ref, sb_ref, g_ref, ca_ref, cb_ref, b_ref, w_ref,
              h_ref, g2_ref, hp2_ref):
    dinv = _dinv_of(ca_ref, cb_ref)
    t = dinv * (sa_ref[...] + sb_ref[...]) + (dinv * dinv) * g_ref[...] + b_ref[...]
    h = jnp.maximum(t, 0.0)
    h_ref[...] = h
    g2 = jnp.dot(h, w_ref[...], preferred_element_type=_f32)
    g2_ref[...] = g2
    hp2_ref[...] = g2 * dinv


def _tc_mid(s1a, s1b, g1, cnt_a, cnt_b, b1, w2, blk):
    n, d = g1.shape
    sds = jax.ShapeDtypeStruct((n, d), _f32)
    return pl.pallas_call(
        _mid_body,
        grid=(n // blk,),
        in_specs=[
            pl.BlockSpec((blk, d), lambda i: (i, 0)),
            pl.BlockSpec((blk, d), lambda i: (i, 0)),
            pl.BlockSpec((blk, d), lambda i: (i, 0)),
            pl.BlockSpec((blk, LANES), lambda i: (i, 0)),
            pl.BlockSpec((blk, LANES), lambda i: (i, 0)),
            pl.BlockSpec((1, d), lambda i: (0, 0)),
            pl.BlockSpec((d, d), lambda i: (0, 0)),
        ],
        out_specs=[
            pl.BlockSpec((blk, d), lambda i: (i, 0)),
            pl.BlockSpec((blk, d), lambda i: (i, 0)),
            pl.BlockSpec((blk, d), lambda i: (i, 0)),
        ],
        out_shape=[sds, sds, sds],
    )(s1a, s1b, g1, cnt_a, cnt_b, b1, w2)


def _post_body(sa_ref, sb_ref, g_ref, ca_ref, cb_ref, b_ref, o_ref):
    dinv = _dinv_of(ca_ref, cb_ref)
    o_ref[...] = (dinv * (sa_ref[...] + sb_ref[...])
                  + (dinv * dinv) * g_ref[...] + b_ref[...])


def _tc_post(s2a, s2b, g2, cnt_a, cnt_b, b2, blk):
    n, d = g2.shape
    return pl.pallas_call(
        _post_body,
        grid=(n // blk,),
        in_specs=[
            pl.BlockSpec((blk, d), lambda i: (i, 0)),
            pl.BlockSpec((blk, d), lambda i: (i, 0)),
            pl.BlockSpec((blk, d), lambda i: (i, 0)),
            pl.BlockSpec((blk, LANES), lambda i: (i, 0)),
            pl.BlockSpec((blk, LANES), lambda i: (i, 0)),
            pl.BlockSpec((1, d), lambda i: (0, 0)),
        ],
        out_specs=pl.BlockSpec((blk, d), lambda i: (i, 0)),
        out_shape=jax.ShapeDtypeStruct((n, d), _f32),
    )(s2a, s2b, g2, cnt_a, cnt_b, b2)


def kernel(x, edge_index, W1, b1, W2, b2):
    n, _ = x.shape
    e = edge_index.shape[1]
    d = W1.shape[1]
    nw = NC * NS
    assert e % (nw * CH) == 0
    k = e // (nw * CH)
    n_pad = ((n + 2047) // 2048) * 2048

    src3 = edge_index[0].reshape(nw, k, CH)
    dst3 = edge_index[1].reshape(nw, k, CH)
    idx4 = jnp.stack([src3, dst3], axis=2)
    zeros = jnp.zeros((n_pad // NS, d), _f32)

    hist = _make_hist(n_pad, k)
    scat = _make_scatter(n, n_pad, d, k)
    blk = 1000

    cnt = hist(dst3)
    g1 = _tc_matmul(x, W1, blk)
    cnt_a = cnt[:n]
    cnt_b = cnt[n_pad:n_pad + n]

    h1p = _tc_pre(cnt_a, cnt_b, g1, blk)
    s1 = scat(h1p, idx4, zeros)
    h1, g2, h2p = _tc_mid(s1[:n], s1[n_pad:n_pad + n], g1, cnt_a, cnt_b,
                          b1.reshape(1, d), W2, blk)
    s2 = scat(h2p, idx4, zeros)
    out = _tc_post(s2[:n], s2[n_pad:n_pad + n], g2, cnt_a, cnt_b,
                   b2.reshape(1, d), blk)
    return (h1, out)


# speedup vs baseline: 1.1191x; 1.1191x over previous
import functools

import jax
import jax.numpy as jnp
from jax import lax
from jax.experimental import pallas as pl
from jax.experimental.pallas import tpu as pltpu
from jax.experimental.pallas import tpu_sc as plsc

NC = 2
NS = 16
LANES = 16
CH = 80

_f32 = jnp.float32


def _sc_mesh():
    return plsc.VectorSubcoreMesh(core_axis_name="c", subcore_axis_name="s")


def _make_hist(n_pad, k):
    stripe = n_pad // NS
    nz = stripe // 128

    @functools.partial(
        pl.kernel,
        out_type=jax.ShapeDtypeStruct((NC * n_pad, LANES), _f32),
        mesh=_sc_mesh(),
        scratch_types=[
            pltpu.VMEM((k, CH), jnp.int32),
            pltpu.VMEM((CH, LANES), _f32),
            pltpu.VMEM((128, LANES), _f32),
            pltpu.VMEM_SHARED((n_pad, LANES), _f32),
            pltpu.SemaphoreType.DMA,
        ],
    )
    def hist(dst_hbm, out_hbm, dst_v, ones_v, z_v, acc, sem):
        cid = lax.axis_index("c")
        sid = lax.axis_index("s")
        wid = cid * NS + sid

        onehot = jnp.where(lax.iota(jnp.int32, LANES) == 0, 1.0, 0.0)
        zrow = jnp.zeros((LANES,), _f32)

        @pl.loop(0, 128)
        def _(r):
            z_v[r] = zrow

        @pl.loop(0, CH)
        def _(r):
            ones_v[r] = onehot

        for t in range(nz):
            pltpu.sync_copy(z_v, acc.at[pl.ds(sid * stripe + t * 128, 128)])
        plsc.subcore_barrier()

        pltpu.async_copy(dst_hbm.at[wid], dst_v, sem).wait()

        fire = 25
        assert k % fire == 0

        @pl.loop(0, k, step=fire)
        def _(j0):
            for t in range(fire):
                pltpu.async_copy(ones_v, acc.at[dst_v.at[j0 + t]], sem, add=True)
            for t in range(fire):
                pltpu.make_async_copy(ones_v, acc.at[dst_v.at[j0 + t]], sem).wait()

        plsc.subcore_barrier()
        pltpu.sync_copy(
            acc.at[pl.ds(sid * stripe, stripe)],
            out_hbm.at[pl.ds(cid * n_pad + sid * stripe, stripe)],
        )

    return hist


def _make_scatter(n, n_pad, d, ng):
    stripe = n_pad // NS
    assert ng % 2 == 0

    @functools.partial(
        pl.kernel,
        out_type=jax.ShapeDtypeStruct((NC * n_pad, d), _f32),
        mesh=_sc_mesh(),
        scratch_types=[
            pltpu.VMEM((3, 2, CH), jnp.int32),
            pltpu.VMEM((3, 2, CH), jnp.int32),
            [pltpu.VMEM((CH, d), _f32) for _ in range(3)],
            pltpu.VMEM_SHARED((n_pad, d), _f32),
            pltpu.SemaphoreType.DMA,
            pltpu.SemaphoreType.DMA,
            [pltpu.SemaphoreType.DMA for _ in range(3)],
            [pltpu.SemaphoreType.DMA for _ in range(3)],
        ],
    )
    def scatter(rows_hbm, idx_hbm, zeros_hbm, out_hbm,
                iba, ibb, bufs, acc, isema, isemb, gsem, ssem):
        cid = lax.axis_index("c")
        sid = lax.axis_index("s")
        wid = cid * NS + sid
        me = idx_hbm.at[wid]

        pltpu.sync_copy(zeros_hbm, acc.at[pl.ds(sid * stripe, stripe)])
        plsc.subcore_barrier()

        pltpu.sync_copy(me.at[0], iba)
        for t in range(3):
            pltpu.async_copy(rows_hbm.at[iba.at[t, 0]], bufs[t], gsem[t])
        pltpu.async_copy(me.at[1], ibb, isemb)

        def half(i, g, ib, isem_this, ib_next, isem_next, last):
            for t in range(3):
                pltpu.make_async_copy(rows_hbm.at[ib.at[t, 0]],
                                      bufs[t], gsem[t]).wait()
                pltpu.async_copy(bufs[t], acc.at[ib.at[t, 1]], ssem[t],
                                 add=True)
            pltpu.make_async_copy(me.at[g + 1], ib_next, isem_next).wait()
            for t in range(3):
                pltpu.make_async_copy(bufs[t], acc.at[ib.at[t, 1]],
                                      ssem[t]).wait()
                pltpu.async_copy(rows_hbm.at[ib_next.at[t, 0]],
                                 bufs[t], gsem[t])
            if last is None:
                pltpu.async_copy(me.at[g + 2], ib, isem_this)
            else:
                @pl.when(last)
                def _():
                    pltpu.async_copy(me.at[g + 2], ib, isem_this)

        @pl.loop(0, ng, step=2)
        def _(g):
            half(g, g, iba, isema, ibb, isemb, None)
            half(g, g + 1, ibb, isemb, iba, isema, g + 3 <= ng)

        for t in range(3):
            pltpu.make_async_copy(rows_hbm.at[iba.at[t, 0]],
                                  bufs[t], gsem[t]).wait()

        plsc.subcore_barrier()
        pltpu.sync_copy(
            acc.at[pl.ds(sid * stripe, stripe)],
            out_hbm.at[pl.ds(cid * n_pad + sid * stripe, stripe)],
        )

    return scatter


def _mm_body(x_ref, w_ref, o_ref):
    o_ref[...] = jnp.dot(x_ref[...], w_ref[...], preferred_element_type=_f32)


def _tc_matmul(x, w, blk):
    n, din = x.shape
    dout = w.shape[1]
    return pl.pallas_call(
        _mm_body,
        grid=(n // blk,),
        in_specs=[
            pl.BlockSpec((blk, din), lambda i: (i, 0)),
            pl.BlockSpec((din, dout), lambda i: (0, 0)),
        ],
        out_specs=pl.BlockSpec((blk, dout), lambda i: (i, 0)),
        out_shape=jax.ShapeDtypeStruct((n, dout), _f32),
    )(x, w)


def _dinv_of(ca_ref, cb_ref):
    deg = 1.0 + ca_ref[..., 0:1] + cb_ref[..., 0:1]
    return lax.rsqrt(deg)


def _pre_body(ca_ref, cb_ref, g_ref, hp_ref):
    hp_ref[...] = g_ref[...] * _dinv_of(ca_ref, cb_ref)


def _tc_pre(cnt_a, cnt_b, g, blk):
    n, d = g.shape
    return pl.pallas_call(
        _pre_body,
        grid=(n // blk,),
        in_specs=[
            pl.BlockSpec((blk, LANES), lambda i: (i, 0)),
            pl.BlockSpec((blk, LANES), lambda i: (i, 0)),
            pl.BlockSpec((blk, d), lambda i: (i, 0)),
        ],
        out_specs=pl.BlockSpec((blk, d), lambda i: (i, 0)),
        out_shape=jax.ShapeDtypeStruct((n, d), _f32),
    )(cnt_a, cnt_b, g)


def _mid_body(sa_ref, sb_ref, g_ref, ca_ref, cb_ref, b_ref, w_ref,
              h_ref, g2_ref, hp2_ref):
    dinv = _dinv_of(ca_ref, cb_ref)
    t = dinv * (sa_ref[...] + sb_ref[...]) + (dinv * dinv) * g_ref[...] + b_ref[...]
    h = jnp.maximum(t, 0.0)
    h_ref[...] = h
    g2 = jnp.dot(h, w_ref[...], preferred_element_type=_f32)
    g2_ref[...] = g2
    hp2_ref[...] = g2 * dinv


def _tc_mid(s1a, s1b, g1, cnt_a, cnt_b, b1, w2, blk):
    n, d = g1.shape
    sds = jax.ShapeDtypeStruct((n, d), _f32)
    return pl.pallas_call(
        _mid_body,
        grid=(n // blk,),
        in_specs=[
            pl.BlockSpec((blk, d), lambda i: (i, 0)),
            pl.BlockSpec((blk, d), lambda i: (i, 0)),
            pl.BlockSpec((blk, d), lambda i: (i, 0)),
            pl.BlockSpec((blk, LANES), lambda i: (i, 0)),
            pl.BlockSpec((blk, LANES), lambda i: (i, 0)),
            pl.BlockSpec((1, d), lambda i: (0, 0)),
            pl.BlockSpec((d, d), lambda i: (0, 0)),
        ],
        out_specs=[
            pl.BlockSpec((blk, d), lambda i: (i, 0)),
            pl.BlockSpec((blk, d), lambda i: (i, 0)),
            pl.BlockSpec((blk, d), lambda i: (i, 0)),
        ],
        out_shape=[sds, sds, sds],
    )(s1a, s1b, g1, cnt_a, cnt_b, b1, w2)


def _post_body(sa_ref, sb_ref, g_ref, ca_ref, cb_ref, b_ref, o_ref):
    dinv = _dinv_of(ca_ref, cb_ref)
    o_ref[...] = (dinv * (sa_ref[...] + sb_ref[...])
                  + (dinv * dinv) * g_ref[...] + b_ref[...])


def _tc_post(s2a, s2b, g2, cnt_a, cnt_b, b2, blk):
    n, d = g2.shape
    return pl.pallas_call(
        _post_body,
        grid=(n // blk,),
        in_specs=[
            pl.BlockSpec((blk, d), lambda i: (i, 0)),
            pl.BlockSpec((blk, d), lambda i: (i, 0)),
            pl.BlockSpec((blk, d), lambda i: (i, 0)),
            pl.BlockSpec((blk, LANES), lambda i: (i, 0)),
            pl.BlockSpec((blk, LANES), lambda i: (i, 0)),
            pl.BlockSpec((1, d), lambda i: (0, 0)),
        ],
        out_specs=pl.BlockSpec((blk, d), lambda i: (i, 0)),
        out_shape=jax.ShapeDtypeStruct((n, d), _f32),
    )(s2a, s2b, g2, cnt_a, cnt_b, b2)


def kernel(x, edge_index, W1, b1, W2, b2):
    n, _ = x.shape
    e = edge_index.shape[1]
    d = W1.shape[1]
    nw = NC * NS
    assert e % (nw * CH) == 0
    k = e // (nw * CH)
    n_pad = ((n + 2047) // 2048) * 2048

    src3 = edge_index[0].reshape(nw, k, CH)
    dst3 = edge_index[1].reshape(nw, k, CH)
    idx4 = jnp.stack([src3, dst3], axis=2)
    zeros = jnp.zeros((n_pad // NS, d), _f32)

    ng = -(-k // 3)
    ng += ng % 2
    n_dummy = 3 * (ng + 1) - k
    src_d = jnp.broadcast_to((jnp.arange(CH) * 127) % n, (nw, n_dummy, CH))
    dst_d = jnp.broadcast_to(n + jnp.arange(CH), (nw, n_dummy, CH))
    dummy = jnp.stack([src_d, dst_d], axis=2).astype(jnp.int32)
    idx5 = jnp.concatenate([idx4, dummy], axis=1).reshape(nw, ng + 1, 3, 2, CH)

    hist = _make_hist(n_pad, k)
    scat = _make_scatter(n, n_pad, d, ng)
    blk = 1000

    cnt = hist(dst3)
    g1 = _tc_matmul(x, W1, blk)
    cnt_a = cnt[:n]
    cnt_b = cnt[n_pad:n_pad + n]

    h1p = _tc_pre(cnt_a, cnt_b, g1, blk)
    s1 = scat(h1p, idx5, zeros)
    h1, g2, h2p = _tc_mid(s1[:n], s1[n_pad:n_pad + n], g1, cnt_a, cnt_b,
                          b1.reshape(1, d), W2, blk)
    s2 = scat(h2p, idx5, zeros)
    out = _tc_post(s2[:n], s2[n_pad:n_pad + n], g2, cnt_a, cnt_b,
                   b2.reshape(1, d), blk)
    return (h1, out)


# speedup vs baseline: 1.1263x; 1.0064x over previous
import functools

import jax
import jax.numpy as jnp
from jax import lax
from jax.experimental import pallas as pl
from jax.experimental.pallas import tpu as pltpu
from jax.experimental.pallas import tpu_sc as plsc

NC = 2
NS = 16
LANES = 16
CH = 80

_f32 = jnp.float32


def _sc_mesh():
    return plsc.VectorSubcoreMesh(core_axis_name="c", subcore_axis_name="s")


def _make_hist(n_pad, k):
    stripe = n_pad // NS
    nz = stripe // 128

    @functools.partial(
        pl.kernel,
        out_type=jax.ShapeDtypeStruct((NC * n_pad, LANES), _f32),
        mesh=_sc_mesh(),
        scratch_types=[
            pltpu.VMEM((k, CH), jnp.int32),
            pltpu.VMEM((CH, LANES), _f32),
            pltpu.VMEM((128, LANES), _f32),
            pltpu.VMEM_SHARED((n_pad, LANES), _f32),
            pltpu.SemaphoreType.DMA,
        ],
    )
    def hist(dst_hbm, out_hbm, dst_v, ones_v, z_v, acc, sem):
        cid = lax.axis_index("c")
        sid = lax.axis_index("s")
        wid = cid * NS + sid

        onehot = jnp.where(lax.iota(jnp.int32, LANES) == 0, 1.0, 0.0)
        zrow = jnp.zeros((LANES,), _f32)

        @pl.loop(0, 128)
        def _(r):
            z_v[r] = zrow

        @pl.loop(0, CH)
        def _(r):
            ones_v[r] = onehot

        for t in range(nz):
            pltpu.sync_copy(z_v, acc.at[pl.ds(sid * stripe + t * 128, 128)])
        plsc.subcore_barrier()

        pltpu.async_copy(dst_hbm.at[wid], dst_v, sem).wait()

        fire = 25
        assert k % fire == 0

        @pl.loop(0, k, step=fire)
        def _(j0):
            for t in range(fire):
                pltpu.async_copy(ones_v, acc.at[dst_v.at[j0 + t]], sem, add=True)
            for t in range(fire):
                pltpu.make_async_copy(ones_v, acc.at[dst_v.at[j0 + t]], sem).wait()

        plsc.subcore_barrier()
        pltpu.sync_copy(
            acc.at[pl.ds(sid * stripe, stripe)],
            out_hbm.at[pl.ds(cid * n_pad + sid * stripe, stripe)],
        )

    return hist


def _make_scatter(n, n_pad, d, ng):
    stripe = n_pad // NS
    assert ng % 2 == 0

    @functools.partial(
        pl.kernel,
        out_type=jax.ShapeDtypeStruct((NC * n_pad, d), _f32),
        mesh=_sc_mesh(),
        scratch_types=[
            pltpu.VMEM((6, CH), jnp.int32),
            pltpu.VMEM((6, CH), jnp.int32),
            [pltpu.VMEM((CH, d), _f32) for _ in range(3)],
            pltpu.VMEM_SHARED((n_pad, d), _f32),
            pltpu.SemaphoreType.DMA,
            pltpu.SemaphoreType.DMA,
            [pltpu.SemaphoreType.DMA for _ in range(3)],
            [pltpu.SemaphoreType.DMA for _ in range(3)],
        ],
    )
    def scatter(rows_hbm, idx_hbm, zeros_hbm, out_hbm,
                iba, ibb, bufs, acc, isema, isemb, gsem, ssem):
        cid = lax.axis_index("c")
        sid = lax.axis_index("s")
        wid = cid * NS + sid
        me = idx_hbm.at[wid]

        pltpu.sync_copy(zeros_hbm, acc.at[pl.ds(sid * stripe, stripe)])
        plsc.subcore_barrier()

        pltpu.sync_copy(me.at[0], iba)
        for t in range(3):
            pltpu.async_copy(rows_hbm.at[iba.at[2 * t]], bufs[t], gsem[t])
        pltpu.async_copy(me.at[1], ibb, isemb)

        def half(i, g, ib, isem_this, ib_next, isem_next, last):
            for t in range(3):
                pltpu.make_async_copy(rows_hbm.at[ib.at[2 * t]],
                                      bufs[t], gsem[t]).wait()
                pltpu.async_copy(bufs[t], acc.at[ib.at[2 * t + 1]], ssem[t],
                                 add=True)
            pltpu.make_async_copy(me.at[g + 1], ib_next, isem_next).wait()
            for t in range(3):
                pltpu.make_async_copy(bufs[t], acc.at[ib.at[2 * t + 1]],
                                      ssem[t]).wait()
                pltpu.async_copy(rows_hbm.at[ib_next.at[2 * t]],
                                 bufs[t], gsem[t])
            if last is None:
                pltpu.async_copy(me.at[g + 2], ib, isem_this)
            else:
                @pl.when(last)
                def _():
                    pltpu.async_copy(me.at[g + 2], ib, isem_this)

        @pl.loop(0, ng, step=2)
        def _(g):
            half(g, g, iba, isema, ibb, isemb, None)
            half(g, g + 1, ibb, isemb, iba, isema, g + 3 <= ng)

        for t in range(3):
            pltpu.make_async_copy(rows_hbm.at[iba.at[2 * t]],
                                  bufs[t], gsem[t]).wait()

        plsc.subcore_barrier()
        pltpu.sync_copy(
            acc.at[pl.ds(sid * stripe, stripe)],
            out_hbm.at[pl.ds(cid * n_pad + sid * stripe, stripe)],
        )

    return scatter


def _mm_body(x_ref, w_ref, o_ref):
    o_ref[...] = jnp.dot(x_ref[...], w_ref[...], preferred_element_type=_f32)


def _tc_matmul(x, w, blk):
    n, din = x.shape
    dout = w.shape[1]
    return pl.pallas_call(
        _mm_body,
        grid=(n // blk,),
        in_specs=[
            pl.BlockSpec((blk, din), lambda i: (i, 0)),
            pl.BlockSpec((din, dout), lambda i: (0, 0)),
        ],
        out_specs=pl.BlockSpec((blk, dout), lambda i: (i, 0)),
        out_shape=jax.ShapeDtypeStruct((n, dout), _f32),
    )(x, w)


def _dinv_of(ca_ref, cb_ref):
    deg = 1.0 + ca_ref[..., 0:1] + cb_ref[..., 0:1]
    return lax.rsqrt(deg)


def _pre_body(ca_ref, cb_ref, g_ref, hp_ref):
    hp_ref[...] = g_ref[...] * _dinv_of(ca_ref, cb_ref)


def _tc_pre(cnt_a, cnt_b, g, blk):
    n, d = g.shape
    return pl.pallas_call(
        _pre_body,
        grid=(n // blk,),
        in_specs=[
            pl.BlockSpec((blk, LANES), lambda i: (i, 0)),
            pl.BlockSpec((blk, LANES), lambda i: (i, 0)),
            pl.BlockSpec((blk, d), lambda i: (i, 0)),
        ],
        out_specs=pl.BlockSpec((blk, d), lambda i: (i, 0)),
        out_shape=jax.ShapeDtypeStruct((n, d), _f32),
    )(cnt_a, cnt_b, g)


def _mid_body(sa_ref, sb_ref, g_ref, ca_ref, cb_ref, b_ref, w_ref,
              h_ref, g2_ref, hp2_ref):
    dinv = _dinv_of(ca_ref, cb_ref)
    t = dinv * (sa_ref[...] + sb_ref[...]) + (dinv * dinv) * g_ref[...] + b_ref[...]
    h = jnp.maximum(t, 0.0)
    h_ref[...] = h
    g2 = jnp.dot(h, w_ref[...], preferred_element_type=_f32)
    g2_ref[...] = g2
    hp2_ref[...] = g2 * dinv


def _tc_mid(s1a, s1b, g1, cnt_a, cnt_b, b1, w2, blk):
    n, d = g1.shape
    sds = jax.ShapeDtypeStruct((n, d), _f32)
    return pl.pallas_call(
        _mid_body,
        grid=(n // blk,),
        in_specs=[
            pl.BlockSpec((blk, d), lambda i: (i, 0)),
            pl.BlockSpec((blk, d), lambda i: (i, 0)),
            pl.BlockSpec((blk, d), lambda i: (i, 0)),
            pl.BlockSpec((blk, LANES), lambda i: (i, 0)),
            pl.BlockSpec((blk, LANES), lambda i: (i, 0)),
            pl.BlockSpec((1, d), lambda i: (0, 0)),
            pl.BlockSpec((d, d), lambda i: (0, 0)),
        ],
        out_specs=[
            pl.BlockSpec((blk, d), lambda i: (i, 0)),
            pl.BlockSpec((blk, d), lambda i: (i, 0)),
            pl.BlockSpec((blk, d), lambda i: (i, 0)),
        ],
        out_shape=[sds, sds, sds],
    )(s1a, s1b, g1, cnt_a, cnt_b, b1, w2)


def _post_body(sa_ref, sb_ref, g_ref, ca_ref, cb_ref, b_ref, o_ref):
    dinv = _dinv_of(ca_ref, cb_ref)
    o_ref[...] = (dinv * (sa_ref[...] + sb_ref[...])
                  + (dinv * dinv) * g_ref[...] + b_ref[...])


def _tc_post(s2a, s2b, g2, cnt_a, cnt_b, b2, blk):
    n, d = g2.shape
    return pl.pallas_call(
        _post_body,
        grid=(n // blk,),
        in_specs=[
            pl.BlockSpec((blk, d), lambda i: (i, 0)),
            pl.BlockSpec((blk, d), lambda i: (i, 0)),
            pl.BlockSpec((blk, d), lambda i: (i, 0)),
            pl.BlockSpec((blk, LANES), lambda i: (i, 0)),
            pl.BlockSpec((blk, LANES), lambda i: (i, 0)),
            pl.BlockSpec((1, d), lambda i: (0, 0)),
        ],
        out_specs=pl.BlockSpec((blk, d), lambda i: (i, 0)),
        out_shape=jax.ShapeDtypeStruct((n, d), _f32),
    )(s2a, s2b, g2, cnt_a, cnt_b, b2)


def kernel(x, edge_index, W1, b1, W2, b2):
    n, _ = x.shape
    e = edge_index.shape[1]
    d = W1.shape[1]
    nw = NC * NS
    assert e % (nw * CH) == 0
    k = e // (nw * CH)
    n_pad = ((n + 2047) // 2048) * 2048

    src3 = edge_index[0].reshape(nw, k, CH)
    dst3 = edge_index[1].reshape(nw, k, CH)
    idx4 = jnp.stack([src3, dst3], axis=2)
    zeros = jnp.zeros((n_pad // NS, d), _f32)

    ng = -(-k // 3)
    ng += ng % 2
    n_dummy = 3 * (ng + 1) - k
    src_d = jnp.broadcast_to((jnp.arange(CH) * 127) % n, (nw, n_dummy, CH))
    dst_d = jnp.broadcast_to(n + jnp.arange(CH), (nw, n_dummy, CH))
    dummy = jnp.stack([src_d, dst_d], axis=2).astype(jnp.int32)
    idx5 = jnp.concatenate([idx4, dummy], axis=1).reshape(nw, ng + 1, 6, CH)

    hist = _make_hist(n_pad, k)
    scat = _make_scatter(n, n_pad, d, ng)
    blk = 1000

    cnt = hist(dst3)
    g1 = _tc_matmul(x, W1, blk)
    cnt_a = cnt[:n]
    cnt_b = cnt[n_pad:n_pad + n]

    h1p = _tc_pre(cnt_a, cnt_b, g1, blk)
    s1 = scat(h1p, idx5, zeros)
    h1, g2, h2p = _tc_mid(s1[:n], s1[n_pad:n_pad + n], g1, cnt_a, cnt_b,
                          b1.reshape(1, d), W2, blk)
    s2 = scat(h2p, idx5, zeros)
    out = _tc_post(s2[:n], s2[n_pad:n_pad + n], g2, cnt_a, cnt_b,
                   b2.reshape(1, d), blk)
    return (h1, out)


# speedup vs baseline: 1.1889x; 1.0556x over previous
import functools

import jax
import jax.numpy as jnp
from jax import lax
from jax.experimental import pallas as pl
from jax.experimental.pallas import tpu as pltpu
from jax.experimental.pallas import tpu_sc as plsc

NC = 2
NS = 16
LANES = 16
CH = 80

_f32 = jnp.float32


def _sc_mesh():
    return plsc.VectorSubcoreMesh(core_axis_name="c", subcore_axis_name="s")


def _make_hist(n_pad, k):
    stripe = n_pad // NS
    nz = stripe // 128

    @functools.partial(
        pl.kernel,
        out_type=jax.ShapeDtypeStruct((NC * n_pad, LANES), _f32),
        mesh=_sc_mesh(),
        scratch_types=[
            pltpu.VMEM((k, CH), jnp.int32),
            pltpu.VMEM((CH, LANES), _f32),
            pltpu.VMEM((128, LANES), _f32),
            pltpu.VMEM_SHARED((n_pad, LANES), _f32),
            pltpu.SemaphoreType.DMA,
        ],
    )
    def hist(dst_hbm, out_hbm, dst_v, ones_v, z_v, acc, sem):
        cid = lax.axis_index("c")
        sid = lax.axis_index("s")
        wid = cid * NS + sid

        onehot = jnp.where(lax.iota(jnp.int32, LANES) == 0, 1.0, 0.0)
        zrow = jnp.zeros((LANES,), _f32)

        @pl.loop(0, 128)
        def _(r):
            z_v[r] = zrow

        @pl.loop(0, CH)
        def _(r):
            ones_v[r] = onehot

        for t in range(nz):
            pltpu.sync_copy(z_v, acc.at[pl.ds(sid * stripe + t * 128, 128)])
        plsc.subcore_barrier()

        pltpu.async_copy(dst_hbm.at[wid], dst_v, sem).wait()

        fire = 25
        assert k % fire == 0

        @pl.loop(0, k, step=fire)
        def _(j0):
            for t in range(fire):
                pltpu.async_copy(ones_v, acc.at[dst_v.at[j0 + t]], sem, add=True)
            for t in range(fire):
                pltpu.make_async_copy(ones_v, acc.at[dst_v.at[j0 + t]], sem).wait()

        plsc.subcore_barrier()
        pltpu.sync_copy(
            acc.at[pl.ds(sid * stripe, stripe)],
            out_hbm.at[pl.ds(cid * n_pad + sid * stripe, stripe)],
        )

    return hist


def _make_scatter(n, n_pad, d, ng):
    stripe = n_pad // NS
    assert ng % 2 == 0

    @functools.partial(
        pl.kernel,
        out_type=jax.ShapeDtypeStruct((NC * n_pad, d), _f32),
        mesh=_sc_mesh(),
        scratch_types=[
            pltpu.VMEM((6, CH), jnp.int32),
            pltpu.VMEM((6, CH), jnp.int32),
            [pltpu.VMEM((CH, d), _f32) for _ in range(3)],
            pltpu.VMEM_SHARED((n_pad, d), _f32),
            pltpu.SemaphoreType.DMA,
            pltpu.SemaphoreType.DMA,
            [pltpu.SemaphoreType.DMA for _ in range(3)],
            [pltpu.SemaphoreType.DMA for _ in range(3)],
        ],
    )
    def scatter(rows_hbm, idx_hbm, zeros_hbm, out_hbm,
                iba, ibb, bufs, acc, isema, isemb, gsem, ssem):
        cid = lax.axis_index("c")
        sid = lax.axis_index("s")
        wid = cid * NS + sid
        me = idx_hbm.at[wid]

        pltpu.sync_copy(zeros_hbm, acc.at[pl.ds(sid * stripe, stripe)])
        plsc.subcore_barrier()

        pltpu.sync_copy(me.at[0], iba)
        for t in range(3):
            pltpu.async_copy(rows_hbm.at[iba.at[2 * t]], bufs[t], gsem[t])
        pltpu.async_copy(me.at[1], ibb, isemb)

        def half(i, g, ib, isem_this, ib_next, isem_next, last):
            for t in range(3):
                pltpu.make_async_copy(rows_hbm.at[ib.at[2 * t]],
                                      bufs[t], gsem[t]).wait()
                pltpu.async_copy(bufs[t], acc.at[ib.at[2 * t + 1]], ssem[t],
                                 add=True)
            pltpu.make_async_copy(me.at[g + 1], ib_next, isem_next).wait()
            for t in range(3):
                pltpu.make_async_copy(bufs[t], acc.at[ib.at[2 * t + 1]],
                                      ssem[t]).wait()
                pltpu.async_copy(rows_hbm.at[ib_next.at[2 * t]],
                                 bufs[t], gsem[t])
            if last is None:
                pltpu.async_copy(me.at[g + 2], ib, isem_this)
            else:
                @pl.when(last)
                def _():
                    pltpu.async_copy(me.at[g + 2], ib, isem_this)

        @pl.loop(0, ng, step=2)
        def _(g):
            half(g, g, iba, isema, ibb, isemb, None)
            half(g, g + 1, ibb, isemb, iba, isema, g + 3 <= ng)

        for t in range(3):
            pltpu.make_async_copy(rows_hbm.at[iba.at[2 * t]],
                                  bufs[t], gsem[t]).wait()

        plsc.subcore_barrier()
        pltpu.sync_copy(
            acc.at[pl.ds(sid * stripe, stripe)],
            out_hbm.at[pl.ds(cid * n_pad + sid * stripe, stripe)],
        )

    return scatter


def _plane0(blk, w):
    return pl.BlockSpec((1, blk, w), lambda i: (0, i, 0))


def _plane1(blk, w):
    return pl.BlockSpec((1, blk, w), lambda i: (1, i, 0))


def _cnt_spec(blk):
    return pl.BlockSpec((2, blk, LANES), lambda i: (0, i, 0))


def _mm_body(x_ref, w_ref, o_ref):
    o_ref[...] = jnp.dot(x_ref[...], w_ref[...], preferred_element_type=_f32)


def _tc_matmul(x, w, blk):
    n, din = x.shape
    dout = w.shape[1]
    return pl.pallas_call(
        _mm_body,
        grid=(n // blk,),
        in_specs=[
            pl.BlockSpec((blk, din), lambda i: (i, 0)),
            pl.BlockSpec((din, dout), lambda i: (0, 0)),
        ],
        out_specs=pl.BlockSpec((blk, dout), lambda i: (i, 0)),
        out_shape=jax.ShapeDtypeStruct((n, dout), _f32),
    )(x, w)


def _dinv_of(c3_ref):
    deg = 1.0 + c3_ref[0, :, 0:1] + c3_ref[1, :, 0:1]
    return lax.rsqrt(deg)


def _pre_body(c3_ref, g_ref, hp_ref):
    hp_ref[...] = g_ref[...] * _dinv_of(c3_ref)


def _tc_pre(cnt3, g, blk):
    n, d = g.shape
    return pl.pallas_call(
        _pre_body,
        grid=(n // blk,),
        in_specs=[
            _cnt_spec(blk),
            pl.BlockSpec((blk, d), lambda i: (i, 0)),
        ],
        out_specs=pl.BlockSpec((blk, d), lambda i: (i, 0)),
        out_shape=jax.ShapeDtypeStruct((n, d), _f32),
    )(cnt3, g)


def _mid_body(sa_ref, sb_ref, g_ref, c3_ref, b_ref, w_ref,
              h_ref, g2_ref, hp2_ref):
    dinv = _dinv_of(c3_ref)
    t = (dinv * (sa_ref[0] + sb_ref[0]) + (dinv * dinv) * g_ref[...]
         + b_ref[...])
    h = jnp.maximum(t, 0.0)
    h_ref[...] = h
    g2 = jnp.dot(h, w_ref[...], preferred_element_type=_f32)
    g2_ref[...] = g2
    hp2_ref[...] = g2 * dinv


def _tc_mid(s13, g1, cnt3, b1, w2, blk):
    n, d = g1.shape
    sds = jax.ShapeDtypeStruct((n, d), _f32)
    return pl.pallas_call(
        _mid_body,
        grid=(n // blk,),
        in_specs=[
            _plane0(blk, d),
            _plane1(blk, d),
            pl.BlockSpec((blk, d), lambda i: (i, 0)),
            _cnt_spec(blk),
            pl.BlockSpec((1, d), lambda i: (0, 0)),
            pl.BlockSpec((d, d), lambda i: (0, 0)),
        ],
        out_specs=[
            pl.BlockSpec((blk, d), lambda i: (i, 0)),
            pl.BlockSpec((blk, d), lambda i: (i, 0)),
            pl.BlockSpec((blk, d), lambda i: (i, 0)),
        ],
        out_shape=[sds, sds, sds],
    )(s13, s13, g1, cnt3, b1, w2)


def _post_body(sa_ref, sb_ref, g_ref, c3_ref, b_ref, o_ref):
    dinv = _dinv_of(c3_ref)
    o_ref[...] = (dinv * (sa_ref[0] + sb_ref[0])
                  + (dinv * dinv) * g_ref[...] + b_ref[...])


def _tc_post(s23, g2, cnt3, b2, blk):
    n, d = g2.shape
    return pl.pallas_call(
        _post_body,
        grid=(n // blk,),
        in_specs=[
            _plane0(blk, d),
            _plane1(blk, d),
            pl.BlockSpec((blk, d), lambda i: (i, 0)),
            _cnt_spec(blk),
            pl.BlockSpec((1, d), lambda i: (0, 0)),
        ],
        out_specs=pl.BlockSpec((blk, d), lambda i: (i, 0)),
        out_shape=jax.ShapeDtypeStruct((n, d), _f32),
    )(s23, s23, g2, cnt3, b2)


def kernel(x, edge_index, W1, b1, W2, b2):
    n, _ = x.shape
    e = edge_index.shape[1]
    d = W1.shape[1]
    nw = NC * NS
    assert e % (nw * CH) == 0
    k = e // (nw * CH)
    n_pad = ((n + 2047) // 2048) * 2048

    src3 = edge_index[0].reshape(nw, k, CH)
    dst3 = edge_index[1].reshape(nw, k, CH)
    idx4 = jnp.stack([src3, dst3], axis=2)
    zeros = jnp.zeros((n_pad // NS, d), _f32)

    ng = -(-k // 3)
    ng += ng % 2
    n_dummy = 3 * (ng + 1) - k
    src_d = jnp.broadcast_to((jnp.arange(CH) * 127) % n, (nw, n_dummy, CH))
    dst_d = jnp.broadcast_to(n + jnp.arange(CH), (nw, n_dummy, CH))
    dummy = jnp.stack([src_d, dst_d], axis=2).astype(jnp.int32)
    idx5 = jnp.concatenate([idx4, dummy], axis=1).reshape(nw, ng + 1, 6, CH)

    hist = _make_hist(n_pad, k)
    scat = _make_scatter(n, n_pad, d, ng)
    blk = 1000

    cnt3 = hist(dst3).reshape(NC, n_pad, LANES)
    g1 = _tc_matmul(x, W1, blk)

    h1p = _tc_pre(cnt3, g1, blk)
    s13 = scat(h1p, idx5, zeros).reshape(NC, n_pad, d)
    h1, g2, h2p = _tc_mid(s13, g1, cnt3, b1.reshape(1, d), W2, blk)
    s23 = scat(h2p, idx5, zeros).reshape(NC, n_pad, d)
    out = _tc_post(s23, g2, cnt3, b2.reshape(1, d), blk)
    return (h1, out)


# speedup vs baseline: 1.1903x; 1.0012x over previous
import functools

import jax
import jax.numpy as jnp
from jax import lax
from jax.experimental import pallas as pl
from jax.experimental.pallas import tpu as pltpu
from jax.experimental.pallas import tpu_sc as plsc

NC = 2
NS = 16
LANES = 16
CH = 80

_f32 = jnp.float32


def _sc_mesh():
    return plsc.VectorSubcoreMesh(core_axis_name="c", subcore_axis_name="s")


def _make_hist(n_pad, k):
    stripe = n_pad // NS
    nz = stripe // 128

    @functools.partial(
        pl.kernel,
        out_type=jax.ShapeDtypeStruct((NC * n_pad, LANES), _f32),
        mesh=_sc_mesh(),
        scratch_types=[
            pltpu.VMEM((k, CH), jnp.int32),
            pltpu.VMEM((CH, LANES), _f32),
            pltpu.VMEM((128, LANES), _f32),
            pltpu.VMEM_SHARED((n_pad, LANES), _f32),
            pltpu.SemaphoreType.DMA,
        ],
    )
    def hist(dst_hbm, out_hbm, dst_v, ones_v, z_v, acc, sem):
        cid = lax.axis_index("c")
        sid = lax.axis_index("s")
        wid = cid * NS + sid

        onehot = jnp.where(lax.iota(jnp.int32, LANES) == 0, 1.0, 0.0)
        zrow = jnp.zeros((LANES,), _f32)

        @pl.loop(0, 128)
        def _(r):
            z_v[r] = zrow

        @pl.loop(0, CH)
        def _(r):
            ones_v[r] = onehot

        for t in range(nz):
            pltpu.sync_copy(z_v, acc.at[pl.ds(sid * stripe + t * 128, 128)])
        plsc.subcore_barrier()

        pltpu.async_copy(dst_hbm.at[wid], dst_v, sem).wait()

        fire = 25
        assert k % fire == 0

        @pl.loop(0, k, step=fire)
        def _(j0):
            for t in range(fire):
                pltpu.async_copy(ones_v, acc.at[dst_v.at[j0 + t]], sem, add=True)
            for t in range(fire):
                pltpu.make_async_copy(ones_v, acc.at[dst_v.at[j0 + t]], sem).wait()

        plsc.subcore_barrier()
        pltpu.sync_copy(
            acc.at[pl.ds(sid * stripe, stripe)],
            out_hbm.at[pl.ds(cid * n_pad + sid * stripe, stripe)],
        )

    return hist


def _make_scatter(n, n_pad, d, ng):
    stripe = n_pad // NS
    assert ng % 2 == 0

    @functools.partial(
        pl.kernel,
        out_type=jax.ShapeDtypeStruct((NC * n_pad, d), _f32),
        mesh=_sc_mesh(),
        scratch_types=[
            pltpu.VMEM((3, CH), jnp.int32),
            pltpu.VMEM((3, CH), jnp.int32),
            pltpu.VMEM((3, CH), jnp.int32),
            pltpu.VMEM((3, CH), jnp.int32),
            [pltpu.VMEM((CH, d), _f32) for _ in range(3)],
            pltpu.VMEM_SHARED((n_pad, d), _f32),
            pltpu.SemaphoreType.DMA,
            pltpu.SemaphoreType.DMA,
            [pltpu.SemaphoreType.DMA for _ in range(3)],
            [pltpu.SemaphoreType.DMA for _ in range(3)],
        ],
    )
    def scatter(rows_hbm, src_hbm, dst_hbm, zeros_hbm, out_hbm,
                sba, dba, sbb, dbb, bufs, acc, isema, isemb, gsem, ssem):
        cid = lax.axis_index("c")
        sid = lax.axis_index("s")
        wid = cid * NS + sid
        mes = src_hbm.at[wid]
        med = dst_hbm.at[wid]

        pltpu.sync_copy(zeros_hbm, acc.at[pl.ds(sid * stripe, stripe)])
        plsc.subcore_barrier()

        pltpu.sync_copy(mes.at[0], sba)
        pltpu.sync_copy(med.at[0], dba)
        for t in range(3):
            pltpu.async_copy(rows_hbm.at[sba.at[t]], bufs[t], gsem[t])
        pltpu.async_copy(mes.at[1], sbb, isemb)
        pltpu.async_copy(med.at[1], dbb, isemb)

        def half(g, sb, db, isem_this, sb_next, db_next, isem_next, last):
            for t in range(3):
                pltpu.make_async_copy(rows_hbm.at[sb.at[t]],
                                      bufs[t], gsem[t]).wait()
                pltpu.async_copy(bufs[t], acc.at[db.at[t]], ssem[t],
                                 add=True)
            pltpu.make_async_copy(mes.at[g + 1], sb_next, isem_next).wait()
            pltpu.make_async_copy(med.at[g + 1], db_next, isem_next).wait()
            for t in range(3):
                pltpu.make_async_copy(bufs[t], acc.at[db.at[t]],
                                      ssem[t]).wait()
                pltpu.async_copy(rows_hbm.at[sb_next.at[t]],
                                 bufs[t], gsem[t])
            if last is None:
                pltpu.async_copy(mes.at[g + 2], sb, isem_this)
                pltpu.async_copy(med.at[g + 2], db, isem_this)
            else:
                @pl.when(last)
                def _():
                    pltpu.async_copy(mes.at[g + 2], sb, isem_this)
                    pltpu.async_copy(med.at[g + 2], db, isem_this)

        @pl.loop(0, ng, step=2)
        def _(g):
            half(g, sba, dba, isema, sbb, dbb, isemb, None)
            half(g + 1, sbb, dbb, isemb, sba, dba, isema, g + 3 <= ng)

        for t in range(3):
            pltpu.make_async_copy(rows_hbm.at[sba.at[t]],
                                  bufs[t], gsem[t]).wait()

        plsc.subcore_barrier()
        pltpu.sync_copy(
            acc.at[pl.ds(sid * stripe, stripe)],
            out_hbm.at[pl.ds(cid * n_pad + sid * stripe, stripe)],
        )

    return scatter


def _pair_spec(blk, w):
    return pl.BlockSpec((2, blk, w), lambda i: (0, i, 0))


def _cnt_spec(blk):
    return pl.BlockSpec((2, blk, LANES), lambda i: (0, i, 0))


def _mm_body(x_ref, w_ref, o_ref):
    o_ref[...] = jnp.dot(x_ref[...], w_ref[...], preferred_element_type=_f32)


def _tc_matmul(x, w, blk):
    n, din = x.shape
    dout = w.shape[1]
    return pl.pallas_call(
        _mm_body,
        grid=(n // blk,),
        in_specs=[
            pl.BlockSpec((blk, din), lambda i: (i, 0)),
            pl.BlockSpec((din, dout), lambda i: (0, 0)),
        ],
        out_specs=pl.BlockSpec((blk, dout), lambda i: (i, 0)),
        out_shape=jax.ShapeDtypeStruct((n, dout), _f32),
    )(x, w)


def _dinv_of(c3_ref):
    deg = 1.0 + c3_ref[0, :, 0:1] + c3_ref[1, :, 0:1]
    return lax.rsqrt(deg)


def _pre_body(c3_ref, g_ref, hp_ref):
    hp_ref[...] = g_ref[...] * _dinv_of(c3_ref)


def _tc_pre(cnt3, g, blk):
    n, d = g.shape
    return pl.pallas_call(
        _pre_body,
        grid=(n // blk,),
        in_specs=[
            _cnt_spec(blk),
            pl.BlockSpec((blk, d), lambda i: (i, 0)),
        ],
        out_specs=pl.BlockSpec((blk, d), lambda i: (i, 0)),
        out_shape=jax.ShapeDtypeStruct((n, d), _f32),
    )(cnt3, g)


def _mid_body(s_ref, hp_ref, c3_ref, b_ref, w_ref, h_ref, hp2_ref):
    dinv = _dinv_of(c3_ref)
    t = dinv * (s_ref[0] + s_ref[1] + hp_ref[...]) + b_ref[...]
    h = jnp.maximum(t, 0.0)
    h_ref[...] = h
    hp2_ref[...] = dinv * jnp.dot(h, w_ref[...], preferred_element_type=_f32)


def _tc_mid(s13, h1p, cnt3, b1, w2, blk):
    n, d = h1p.shape
    sds = jax.ShapeDtypeStruct((n, d), _f32)
    return pl.pallas_call(
        _mid_body,
        grid=(n // blk,),
        in_specs=[
            _pair_spec(blk, d),
            pl.BlockSpec((blk, d), lambda i: (i, 0)),
            _cnt_spec(blk),
            pl.BlockSpec((1, d), lambda i: (0, 0)),
            pl.BlockSpec((d, d), lambda i: (0, 0)),
        ],
        out_specs=[
            pl.BlockSpec((blk, d), lambda i: (i, 0)),
            pl.BlockSpec((blk, d), lambda i: (i, 0)),
        ],
        out_shape=[sds, sds],
    )(s13, h1p, cnt3, b1, w2)


def _post_body(s_ref, hp_ref, c3_ref, b_ref, o_ref):
    dinv = _dinv_of(c3_ref)
    o_ref[...] = dinv * (s_ref[0] + s_ref[1] + hp_ref[...]) + b_ref[...]


def _tc_post(s23, h2p, cnt3, b2, blk):
    n, d = h2p.shape
    return pl.pallas_call(
        _post_body,
        grid=(n // blk,),
        in_specs=[
            _pair_spec(blk, d),
            pl.BlockSpec((blk, d), lambda i: (i, 0)),
            _cnt_spec(blk),
            pl.BlockSpec((1, d), lambda i: (0, 0)),
        ],
        out_specs=pl.BlockSpec((blk, d), lambda i: (i, 0)),
        out_shape=jax.ShapeDtypeStruct((n, d), _f32),
    )(s23, h2p, cnt3, b2)


def kernel(x, edge_index, W1, b1, W2, b2):
    n, _ = x.shape
    e = edge_index.shape[1]
    d = W1.shape[1]
    nw = NC * NS
    assert e % (nw * CH) == 0
    k = e // (nw * CH)
    n_pad = ((n + 2047) // 2048) * 2048

    src3 = edge_index[0].reshape(nw, k, CH)
    dst3 = edge_index[1].reshape(nw, k, CH)
    zeros = jnp.zeros((n_pad // NS, d), _f32)

    ng = -(-k // 3)
    ng += ng % 2
    n_dummy = 3 * (ng + 1) - k
    src_d = jnp.broadcast_to((jnp.arange(CH) * 127) % n,
                             (nw, n_dummy, CH)).astype(jnp.int32)
    dst_d = jnp.broadcast_to(n + jnp.arange(CH),
                             (nw, n_dummy, CH)).astype(jnp.int32)
    src5 = jnp.concatenate([src3, src_d], axis=1).reshape(nw, ng + 1, 3, CH)
    dst5 = jnp.concatenate([dst3, dst_d], axis=1).reshape(nw, ng + 1, 3, CH)

    hist = _make_hist(n_pad, k)
    scat = _make_scatter(n, n_pad, d, ng)
    blk = 1000

    cnt3 = hist(dst3).reshape(NC, n_pad, LANES)
    g1 = _tc_matmul(x, W1, blk)

    h1p = _tc_pre(cnt3, g1, blk)
    s13 = scat(h1p, src5, dst5, zeros).reshape(NC, n_pad, d)
    h1, h2p = _tc_mid(s13, h1p, cnt3, b1.reshape(1, d), W2, blk)
    s23 = scat(h2p, src5, dst5, zeros).reshape(NC, n_pad, d)
    out = _tc_post(s23, h2p, cnt3, b2.reshape(1, d), blk)
    return (h1, out)
